# Initial kernel scaffold; baseline (speedup 1.0000x reference)
import jax
import jax.numpy as jnp
from jax.experimental import pallas as pl


def kernel(input, embed):
    raise NotImplementedError("write your pallas kernel here")


# speedup vs baseline: 1.1413x; 1.1413x over previous
import jax
import jax.numpy as jnp
from jax.experimental import pallas as pl

SEQ = 8192
DIM = 1024
BLOCK_ROWS = 1024


def _copy_body(src_ref, dst_ref):
    dst_ref[...] = src_ref[...]


def kernel(input, embed):
    out = pl.pallas_call(
        _copy_body,
        out_shape=jax.ShapeDtypeStruct((SEQ, DIM), embed.dtype),
        grid=(SEQ // BLOCK_ROWS,),
        in_specs=[pl.BlockSpec((BLOCK_ROWS, DIM), lambda i: (i, 0))],
        out_specs=pl.BlockSpec((BLOCK_ROWS, DIM), lambda i: (i, 0)),
    )(embed)
    return out.reshape(SEQ, 1, DIM)
